# baseline (speedup 1.0000x reference)
import functools

import jax
import jax.numpy as jnp
from jax import lax
from jax.experimental import pallas as pl
from jax.experimental.pallas import tpu as pltpu
from jax.experimental.pallas import tpu_sc as plsc

B = 16384
D = 64
W = D // 2
L = 16
NW = 32
RPW = B // NW
CH = 128
NCH = RPW // CH
MARGIN = 1.0

_mesh = plsc.VectorSubcoreMesh(core_axis_name="c", subcore_axis_name="s")


@functools.partial(
    pl.kernel,
    out_type=[
        jax.ShapeDtypeStruct((B, L), jnp.float32),
        jax.ShapeDtypeStruct((B, L), jnp.float32),
    ],
    mesh=_mesh,
    compiler_params=pltpu.CompilerParams(use_tc_tiling_on_sc=False),
    scratch_types=[
        pltpu.VMEM((6, NCH, CH), jnp.int32),
        pltpu.VMEM((RPW, W), jnp.int32),
        pltpu.VMEM((RPW, W), jnp.int32),
        pltpu.VMEM((RPW, W), jnp.int32),
        pltpu.VMEM((RPW, L), jnp.float32),
        pltpu.SemaphoreType.DMA,
    ],
)
def _sc_scores(idx_hbm, ent_hbm, rel_hbm, pos_out, neg_out,
               idx_v, h_v, r_v, t_v, part_v, sem):
    wid = lax.axis_index("s") * 2 + lax.axis_index("c")
    base = wid * RPW
    pltpu.sync_copy(idx_hbm.at[wid], idx_v)

    for side, out_hbm in ((0, pos_out), (1, neg_out)):
        copies = []
        for j in range(NCH):
            dst = pl.ds(j * CH, CH)
            copies.append(pltpu.async_copy(
                ent_hbm.at[idx_v.at[3 * side + 0, j]], h_v.at[dst], sem))
            copies.append(pltpu.async_copy(
                rel_hbm.at[idx_v.at[3 * side + 1, j]], r_v.at[dst], sem))
            copies.append(pltpu.async_copy(
                ent_hbm.at[idx_v.at[3 * side + 2, j]], t_v.at[dst], sem))
        for cp in copies:
            cp.wait()

        def dist(g, carry):
            for j in range(L):
                i = g * L + j
                s = None
                for k in range(W // L):
                    dsl = pl.ds(k * L, L)
                    hw = h_v[i, dsl]
                    tw = t_v[i, dsl]
                    rw = r_v[i, dsl]
                    h0 = jax.lax.bitcast_convert_type(hw << 16, jnp.float32)
                    t0 = jax.lax.bitcast_convert_type(tw << 16, jnp.float32)
                    r0 = jax.lax.bitcast_convert_type(rw << 16, jnp.float32)
                    hi_mask = jnp.full((L,), -65536, jnp.int32)
                    h1 = jax.lax.bitcast_convert_type(hw & hi_mask, jnp.float32)
                    t1 = jax.lax.bitcast_convert_type(tw & hi_mask, jnp.float32)
                    r1 = jax.lax.bitcast_convert_type(rw & hi_mask, jnp.float32)
                    d0 = h0 + r0 - t0
                    d1 = h1 + r1 - t1
                    sq = d0 * d0 + d1 * d1
                    s = sq if s is None else s + sq
                part_v[i, :] = s
            return carry

        lax.fori_loop(0, RPW // L, dist, 0)

        pltpu.sync_copy(part_v, out_hbm.at[pl.ds(base, RPW)])


def _tc_loss(p_ref, n_ref, o_ref):
    sp = jnp.sqrt(jnp.sum(p_ref[...], axis=1))
    sn = jnp.sqrt(jnp.sum(n_ref[...], axis=1))
    hinge = jnp.maximum(MARGIN + sp - sn, 0.0)
    o_ref[0] = jnp.sum(hinge) * (1.0 / B)


_loss_call = pl.pallas_call(
    _tc_loss,
    out_shape=jax.ShapeDtypeStruct((1,), jnp.float32),
    out_specs=pl.BlockSpec(memory_space=pltpu.SMEM),
)


def _pack_bf16_pairs(table):
    bits = jax.lax.bitcast_convert_type(table, jnp.uint32)
    lo = bits[:, 0::2] >> 16
    hi = bits[:, 1::2] & jnp.uint32(0xFFFF0000)
    return jax.lax.bitcast_convert_type(hi | lo, jnp.int32)


def kernel(pos_triples, neg_triples, entity_emb, relation_emb):
    pt = pos_triples.astype(jnp.int32)
    nt = neg_triples.astype(jnp.int32)
    idx = jnp.stack(
        [pt[:, 0], pt[:, 1], pt[:, 2], nt[:, 0], nt[:, 1], nt[:, 2]], axis=0)
    idx = idx.reshape(6, NW, NCH, CH).transpose(1, 0, 2, 3)
    sq_pos, sq_neg = _sc_scores(
        idx, _pack_bf16_pairs(entity_emb), _pack_bf16_pairs(relation_emb))
    loss = _loss_call(sq_pos, sq_neg)
    return loss[0]


# speedup vs baseline: 35.9854x; 35.9854x over previous
import functools

import jax
import jax.numpy as jnp
from jax import lax
from jax.experimental import pallas as pl
from jax.experimental.pallas import tpu as pltpu
from jax.experimental.pallas import tpu_sc as plsc

B = 16384
D = 64
L = 16
NW = 32
RPW = B // NW
CH = 128
NCH = RPW // CH
MARGIN = 1.0

_mesh = plsc.VectorSubcoreMesh(core_axis_name="c", subcore_axis_name="s")


@functools.partial(
    pl.kernel,
    out_type=[
        jax.ShapeDtypeStruct((B, L), jnp.float32),
        jax.ShapeDtypeStruct((B, L), jnp.float32),
    ],
    mesh=_mesh,
    scratch_types=[
        pltpu.VMEM((6 * NCH, CH), jnp.int32),
        pltpu.VMEM((CH, D), jnp.float32),
        pltpu.VMEM((CH, D), jnp.float32),
        pltpu.VMEM((CH, D), jnp.float32),
        pltpu.VMEM((CH, L), jnp.float32),
        pltpu.SemaphoreType.DMA,
    ],
)
def _sc_scores(idx_hbm, ent_hbm, rel_hbm, pos_out, neg_out,
               idx_v, h_v, r_v, t_v, part_v, sem):
    wid = lax.axis_index("s") * 2 + lax.axis_index("c")
    base = wid * RPW
    pltpu.sync_copy(idx_hbm.at[wid], idx_v)

    for side, out_hbm in ((0, pos_out), (1, neg_out)):
        for c in range(NCH):
            row_h = (3 * side + 0) * NCH + c
            row_r = (3 * side + 1) * NCH + c
            row_t = (3 * side + 2) * NCH + c

            def fire(g, carry):
                gsl = pl.ds(g * L, L)
                hv = idx_v[row_h, gsl]
                rv = idx_v[row_r, gsl]
                tv = idx_v[row_t, gsl]
                for j in range(L):
                    i = g * L + j
                    pltpu.async_copy(
                        ent_hbm.at[hv[j] >> 3, hv[j] & 7], h_v.at[i], sem)
                    pltpu.async_copy(rel_hbm.at[rv[j]], r_v.at[i], sem)
                    pltpu.async_copy(
                        ent_hbm.at[tv[j] >> 3, tv[j] & 7], t_v.at[i], sem)
                return carry

            lax.fori_loop(0, CH // L, fire, 0)
            pltpu.make_async_copy(rel_hbm.at[pl.ds(0, CH)], h_v, sem).wait()
            pltpu.make_async_copy(rel_hbm.at[pl.ds(0, CH)], r_v, sem).wait()
            pltpu.make_async_copy(rel_hbm.at[pl.ds(0, CH)], t_v, sem).wait()

            def dist(i, carry):
                s = None
                for k in range(D // L):
                    dsl = pl.ds(k * L, L)
                    dv = h_v[i, dsl] + r_v[i, dsl] - t_v[i, dsl]
                    sq = dv * dv
                    s = sq if s is None else s + sq
                part_v[i, :] = s
                return carry

            lax.fori_loop(0, CH, dist, 0)

            pltpu.sync_copy(part_v, out_hbm.at[pl.ds(base + c * CH, CH)])


def _tc_loss(p_ref, n_ref, o_ref):
    sp = jnp.sqrt(jnp.sum(p_ref[...], axis=1))
    sn = jnp.sqrt(jnp.sum(n_ref[...], axis=1))
    hinge = jnp.maximum(MARGIN + sp - sn, 0.0)
    o_ref[0] = jnp.sum(hinge) * (1.0 / B)


_loss_call = pl.pallas_call(
    _tc_loss,
    out_shape=jax.ShapeDtypeStruct((1,), jnp.float32),
    out_specs=pl.BlockSpec(memory_space=pltpu.SMEM),
)


def kernel(pos_triples, neg_triples, entity_emb, relation_emb):
    pt = pos_triples.astype(jnp.int32)
    nt = neg_triples.astype(jnp.int32)
    idx = jnp.stack(
        [pt[:, 0], pt[:, 1], pt[:, 2], nt[:, 0], nt[:, 1], nt[:, 2]], axis=0)
    idx = idx.reshape(6, NW, NCH, CH).transpose(1, 0, 2, 3)
    idx = idx.reshape(NW, 6 * NCH, CH)
    ent3 = entity_emb.reshape(125000, 8, D)
    sq_pos, sq_neg = _sc_scores(idx, ent3, relation_emb)
    loss = _loss_call(sq_pos, sq_neg)
    return loss[0]


# speedup vs baseline: 37.3135x; 1.0369x over previous
import functools

import jax
import jax.numpy as jnp
from jax import lax
from jax.experimental import pallas as pl
from jax.experimental.pallas import tpu as pltpu
from jax.experimental.pallas import tpu_sc as plsc

B = 16384
D = 64
L = 16
NW = 32
RPW = B // NW
CH = 128
NCH = RPW // CH
MARGIN = 1.0

_mesh = plsc.VectorSubcoreMesh(core_axis_name="c", subcore_axis_name="s")


@functools.partial(
    pl.kernel,
    out_type=[
        jax.ShapeDtypeStruct((B, L), jnp.float32),
        jax.ShapeDtypeStruct((B, L), jnp.float32),
    ],
    mesh=_mesh,
    scratch_types=[
        pltpu.VMEM((6 * NCH, CH), jnp.int32),
        pltpu.VMEM((2, CH, D), jnp.float32),
        pltpu.VMEM((2, CH, D), jnp.float32),
        pltpu.VMEM((2, CH, D), jnp.float32),
        pltpu.VMEM((CH, L), jnp.float32),
        pltpu.SemaphoreType.DMA,
        pltpu.SemaphoreType.DMA,
    ],
)
def _sc_scores(idx_hbm, ent_hbm, rel_hbm, pos_out, neg_out,
               idx_v, h_v, r_v, t_v, part_v, sem0, sem1):
    wid = lax.axis_index("s") * 2 + lax.axis_index("c")
    base = wid * RPW
    pltpu.sync_copy(idx_hbm.at[wid], idx_v)

    sems = (sem0, sem1)
    NC2 = 2 * NCH

    def fire(cc):
        side, c = cc // NCH, cc % NCH
        p = cc % 2
        row_h = (3 * side + 0) * NCH + c
        row_r = (3 * side + 1) * NCH + c
        row_t = (3 * side + 2) * NCH + c
        sem = sems[p]

        def body(g, carry):
            gsl = pl.ds(g * L, L)
            hv = idx_v[row_h, gsl]
            rv = idx_v[row_r, gsl]
            tv = idx_v[row_t, gsl]
            for j in range(L):
                i = g * L + j
                pltpu.async_copy(
                    ent_hbm.at[hv[j] >> 3, hv[j] & 7], h_v.at[p, i], sem)
                pltpu.async_copy(rel_hbm.at[rv[j]], r_v.at[p, i], sem)
                pltpu.async_copy(
                    ent_hbm.at[tv[j] >> 3, tv[j] & 7], t_v.at[p, i], sem)
            return carry

        lax.fori_loop(0, CH // L, body, 0)

    def drain_compute_out(cc):
        side, c = cc // NCH, cc % NCH
        p = cc % 2
        sem = sems[p]
        pltpu.make_async_copy(rel_hbm.at[pl.ds(0, CH)], h_v.at[p], sem).wait()
        pltpu.make_async_copy(rel_hbm.at[pl.ds(0, CH)], r_v.at[p], sem).wait()
        pltpu.make_async_copy(rel_hbm.at[pl.ds(0, CH)], t_v.at[p], sem).wait()

        def dist(i, carry):
            s = None
            for k in range(D // L):
                dsl = pl.ds(k * L, L)
                dv = h_v[p, i, dsl] + r_v[p, i, dsl] - t_v[p, i, dsl]
                sq = dv * dv
                s = sq if s is None else s + sq
            part_v[i, :] = s
            return carry

        lax.fori_loop(0, CH, dist, 0)
        out_hbm = pos_out if side == 0 else neg_out
        pltpu.sync_copy(part_v, out_hbm.at[pl.ds(base + c * CH, CH)])

    fire(0)
    for cc in range(NC2):
        if cc + 1 < NC2:
            fire(cc + 1)
        drain_compute_out(cc)


def _tc_loss(p_ref, n_ref, o_ref):
    sp = jnp.sqrt(jnp.sum(p_ref[...], axis=1))
    sn = jnp.sqrt(jnp.sum(n_ref[...], axis=1))
    hinge = jnp.maximum(MARGIN + sp - sn, 0.0)
    o_ref[0] = jnp.sum(hinge) * (1.0 / B)


_loss_call = pl.pallas_call(
    _tc_loss,
    out_shape=jax.ShapeDtypeStruct((1,), jnp.float32),
    out_specs=pl.BlockSpec(memory_space=pltpu.SMEM),
)


def kernel(pos_triples, neg_triples, entity_emb, relation_emb):
    pt = pos_triples.astype(jnp.int32)
    nt = neg_triples.astype(jnp.int32)
    idx = jnp.stack(
        [pt[:, 0], pt[:, 1], pt[:, 2], nt[:, 0], nt[:, 1], nt[:, 2]], axis=0)
    idx = idx.reshape(6, NW, NCH, CH).transpose(1, 0, 2, 3)
    idx = idx.reshape(NW, 6 * NCH, CH)
    ent3 = entity_emb.reshape(125000, 8, D)
    sq_pos, sq_neg = _sc_scores(idx, ent3, relation_emb)
    loss = _loss_call(sq_pos, sq_neg)
    return loss[0]


# speedup vs baseline: 37.9686x; 1.0176x over previous
import functools

import jax
import jax.numpy as jnp
from jax import lax
from jax.experimental import pallas as pl
from jax.experimental.pallas import tpu as pltpu
from jax.experimental.pallas import tpu_sc as plsc

B = 16384
D = 64
L = 16
NW = 32
RPW = B // NW
CH = 128
NCH = RPW // CH
MARGIN = 1.0

_mesh = plsc.VectorSubcoreMesh(core_axis_name="c", subcore_axis_name="s")


@functools.partial(
    pl.kernel,
    out_type=[
        jax.ShapeDtypeStruct((B // 8, 128), jnp.float32),
        jax.ShapeDtypeStruct((B // 8, 128), jnp.float32),
    ],
    mesh=_mesh,
    scratch_types=[
        pltpu.VMEM((6 * NCH, CH), jnp.int32),
        pltpu.VMEM((2, CH, D), jnp.float32),
        pltpu.VMEM((2, CH, D), jnp.float32),
        pltpu.VMEM((2, CH, D), jnp.float32),
        pltpu.VMEM((CH // 8, 128), jnp.float32),
        pltpu.SemaphoreType.DMA,
        pltpu.SemaphoreType.DMA,
    ],
)
def _sc_scores(idx_hbm, ent_hbm, rel_hbm, pos_out, neg_out,
               idx_v, h_v, r_v, t_v, part_v, sem0, sem1):
    wid = lax.axis_index("s") * 2 + lax.axis_index("c")
    base = wid * RPW
    pltpu.sync_copy(idx_hbm.at[wid], idx_v)

    sems = (sem0, sem1)
    NC2 = 2 * NCH

    def fire(cc):
        side, c = cc // NCH, cc % NCH
        p = cc % 2
        row_h = (3 * side + 0) * NCH + c
        row_r = (3 * side + 1) * NCH + c
        row_t = (3 * side + 2) * NCH + c
        sem = sems[p]

        def body(g, carry):
            gsl = pl.ds(g * L, L)
            hv = idx_v[row_h, gsl]
            rv = idx_v[row_r, gsl]
            tv = idx_v[row_t, gsl]
            for j in range(L):
                i = g * L + j
                pltpu.async_copy(
                    ent_hbm.at[hv[j] >> 3, hv[j] & 7], h_v.at[p, i], sem)
                pltpu.async_copy(rel_hbm.at[rv[j]], r_v.at[p, i], sem)
                pltpu.async_copy(
                    ent_hbm.at[tv[j] >> 3, tv[j] & 7], t_v.at[p, i], sem)
            return carry

        lax.fori_loop(0, CH // L, body, 0)

    def drain_compute_out(cc):
        side, c = cc // NCH, cc % NCH
        p = cc % 2
        sem = sems[p]
        pltpu.make_async_copy(rel_hbm.at[pl.ds(0, CH)], h_v.at[p], sem).wait()
        pltpu.make_async_copy(rel_hbm.at[pl.ds(0, CH)], r_v.at[p], sem).wait()
        pltpu.make_async_copy(rel_hbm.at[pl.ds(0, CH)], t_v.at[p], sem).wait()

        def dist(g, carry):
            for m in range(8):
                i = g * 8 + m
                s = None
                for k in range(D // L):
                    dsl = pl.ds(k * L, L)
                    dv = h_v[p, i, dsl] + r_v[p, i, dsl] - t_v[p, i, dsl]
                    sq = dv * dv
                    s = sq if s is None else s + sq
                part_v[g, pl.ds(m * L, L)] = s
            return carry

        lax.fori_loop(0, CH // 8, dist, 0)
        out_hbm = pos_out if side == 0 else neg_out
        off = pl.multiple_of((base + c * CH) // 8, 8)
        pltpu.sync_copy(part_v, out_hbm.at[pl.ds(off, CH // 8)])

    fire(0)
    for cc in range(NC2):
        if cc + 1 < NC2:
            fire(cc + 1)
        drain_compute_out(cc)


def _tc_loss(p_ref, n_ref, o_ref):
    ridx = lax.broadcasted_iota(jnp.int32, (128, 8), 0)
    cidx = lax.broadcasted_iota(jnp.int32, (128, 8), 1)
    seg = jnp.where(ridx // L == cidx, 1.0, 0.0)
    sp = jnp.sqrt(jax.lax.dot(p_ref[...], seg))
    sn = jnp.sqrt(jax.lax.dot(n_ref[...], seg))
    hinge = jnp.maximum(MARGIN + sp - sn, 0.0)
    o_ref[0] = jnp.sum(hinge) * (1.0 / B)


_loss_call = pl.pallas_call(
    _tc_loss,
    out_shape=jax.ShapeDtypeStruct((1,), jnp.float32),
    out_specs=pl.BlockSpec(memory_space=pltpu.SMEM),
)


def kernel(pos_triples, neg_triples, entity_emb, relation_emb):
    pt = pos_triples.astype(jnp.int32)
    nt = neg_triples.astype(jnp.int32)
    idx = jnp.stack(
        [pt[:, 0], pt[:, 1], pt[:, 2], nt[:, 0], nt[:, 1], nt[:, 2]], axis=0)
    idx = idx.reshape(6, NW, NCH, CH).transpose(1, 0, 2, 3)
    idx = idx.reshape(NW, 6 * NCH, CH)
    ent3 = entity_emb.reshape(125000, 8, D)
    sq_pos, sq_neg = _sc_scores(idx, ent3, relation_emb)
    loss = _loss_call(sq_pos, sq_neg)
    return loss[0]


# speedup vs baseline: 38.0560x; 1.0023x over previous
import functools

import jax
import jax.numpy as jnp
from jax import lax
from jax.experimental import pallas as pl
from jax.experimental.pallas import tpu as pltpu
from jax.experimental.pallas import tpu_sc as plsc

B = 16384
D = 64
NR = 1000
L = 16
NW = 32
RPW = B // NW
CH = 128
NCH = RPW // CH
MARGIN = 1.0

_mesh = plsc.VectorSubcoreMesh(core_axis_name="c", subcore_axis_name="s")


@functools.partial(
    pl.kernel,
    out_type=[
        jax.ShapeDtypeStruct((B // 8, 128), jnp.float32),
        jax.ShapeDtypeStruct((B // 8, 128), jnp.float32),
    ],
    mesh=_mesh,
    scratch_types=[
        pltpu.VMEM((6 * NCH, CH), jnp.int32),
        pltpu.VMEM((NR * (D // 2) // 128, 128), jnp.int32),
        pltpu.VMEM((2, CH, D), jnp.float32),
        pltpu.VMEM((2, CH, D), jnp.float32),
        pltpu.VMEM((CH // 8, 128), jnp.float32),
        pltpu.SemaphoreType.DMA,
        pltpu.SemaphoreType.DMA,
        pltpu.SemaphoreType.DMA,
    ],
)
def _sc_scores(idx_hbm, ent_hbm, rel_hbm, relf_hbm, pos_out, neg_out,
               idx_v, rel_v, h_v, t_v, part_v, sem0, sem1, rsem):
    wid = lax.axis_index("s") * 2 + lax.axis_index("c")
    base = wid * RPW
    pltpu.sync_copy(idx_hbm.at[wid], idx_v)
    rel_cp = pltpu.async_copy(rel_hbm, rel_v, rsem)

    sems = (sem0, sem1)
    NC2 = 2 * NCH

    def fire(cc):
        side, c = cc // NCH, cc % NCH
        p = cc % 2
        row_h = (3 * side + 0) * NCH + c
        row_t = (3 * side + 2) * NCH + c
        sem = sems[p]

        def body(g, carry):
            gsl = pl.ds(g * L, L)
            hv = idx_v[row_h, gsl]
            tv = idx_v[row_t, gsl]
            hb, hs = hv >> 3, hv & 7
            tb, ts = tv >> 3, tv & 7
            for j in range(L):
                i = g * L + j
                pltpu.async_copy(
                    ent_hbm.at[hb[j], hs[j]], h_v.at[p, i], sem)
                pltpu.async_copy(
                    ent_hbm.at[tb[j], ts[j]], t_v.at[p, i], sem)
            return carry

        lax.fori_loop(0, CH // L, body, 0)

    def drain_compute_out(cc):
        side, c = cc // NCH, cc % NCH
        p = cc % 2
        sem = sems[p]
        row_r = (3 * side + 1) * NCH + c
        pltpu.make_async_copy(
            relf_hbm.at[pl.ds(0, CH)], h_v.at[p], sem).wait()
        pltpu.make_async_copy(
            relf_hbm.at[pl.ds(0, CH)], t_v.at[p], sem).wait()

        def dist(gg, carry):
            rv = idx_v[row_r, pl.ds(gg * L, L)]
            for m in range(L):
                i = gg * L + m
                q, mm = 2 * gg + m // 8, m % 8
                ri = rv[m]
                s = None
                for k2 in range(D // (2 * L)):
                    rwi = 2 * ri + k2
                    rw = rel_v[rwi >> 3, pl.ds((rwi & 7) * L, L)]
                    r0 = jax.lax.bitcast_convert_type(rw << 16, jnp.float32)
                    r1 = jax.lax.bitcast_convert_type(
                        rw & jnp.full((L,), -65536, jnp.int32), jnp.float32)
                    for k, r in ((2 * k2, r0), (2 * k2 + 1, r1)):
                        dsl = pl.ds(k * L, L)
                        dv = h_v[p, i, dsl] + r - t_v[p, i, dsl]
                        sq = dv * dv
                        s = sq if s is None else s + sq
                part_v[q, pl.ds(mm * L, L)] = s
            return carry

        lax.fori_loop(0, CH // L, dist, 0)
        out_hbm = pos_out if side == 0 else neg_out
        off = pl.multiple_of((base + c * CH) // 8, 8)
        pltpu.sync_copy(part_v, out_hbm.at[pl.ds(off, CH // 8)])

    fire(0)
    rel_cp.wait()
    for cc in range(NC2):
        if cc + 1 < NC2:
            fire(cc + 1)
        drain_compute_out(cc)


def _tc_loss(p_ref, n_ref, o_ref):
    ridx = lax.broadcasted_iota(jnp.int32, (128, 8), 0)
    cidx = lax.broadcasted_iota(jnp.int32, (128, 8), 1)
    seg = jnp.where(ridx // L == cidx, 1.0, 0.0)
    sp = jnp.sqrt(jax.lax.dot(p_ref[...], seg))
    sn = jnp.sqrt(jax.lax.dot(n_ref[...], seg))
    hinge = jnp.maximum(MARGIN + sp - sn, 0.0)
    o_ref[0] = jnp.sum(hinge) * (1.0 / B)


_loss_call = pl.pallas_call(
    _tc_loss,
    out_shape=jax.ShapeDtypeStruct((1,), jnp.float32),
    out_specs=pl.BlockSpec(memory_space=pltpu.SMEM),
)


def _pack_rel_bf16(table):
    bits = jax.lax.bitcast_convert_type(table, jnp.uint32)
    w = jnp.arange(D // 2)
    lo_cols = 32 * (w // L) + (w % L)
    lo = bits[:, lo_cols] >> 16
    hi = bits[:, lo_cols + L] & jnp.uint32(0xFFFF0000)
    packed = jax.lax.bitcast_convert_type(hi | lo, jnp.int32)
    return packed.reshape(NR * (D // 2) // 128, 128)


def kernel(pos_triples, neg_triples, entity_emb, relation_emb):
    pt = pos_triples.astype(jnp.int32)
    nt = neg_triples.astype(jnp.int32)
    idx = jnp.stack(
        [pt[:, 0], pt[:, 1], pt[:, 2], nt[:, 0], nt[:, 1], nt[:, 2]], axis=0)
    idx = idx.reshape(6, NW, NCH, CH).transpose(1, 0, 2, 3)
    idx = idx.reshape(NW, 6 * NCH, CH)
    ent3 = entity_emb.reshape(125000, 8, D)
    sq_pos, sq_neg = _sc_scores(
        idx, ent3, _pack_rel_bf16(relation_emb), relation_emb)
    loss = _loss_call(sq_pos, sq_neg)
    return loss[0]
